# Initial kernel scaffold; baseline (speedup 1.0000x reference)
import jax
import jax.numpy as jnp
from jax.experimental import pallas as pl


def kernel(boxes, gt_boxes):
    raise NotImplementedError("write your pallas kernel here")


# speedup vs baseline: 2.3811x; 2.3811x over previous
import functools

import jax
import jax.numpy as jnp
from jax import lax
from jax.experimental import pallas as pl
from jax.experimental.pallas import tpu as pltpu
from jax.experimental.pallas import tpu_sc as plsc

_ITERATIONS = 5
_STAGES = _ITERATIONS + 1
_LR_POS = 0.45
_LR_SIZE = 0.4
_L = 16


@functools.lru_cache(maxsize=None)
def _make_sc_call(B, N, G):
    assert B == 2, "kernel specialized to B == 2"
    assert N % _L == 0 and G % _L == 0

    info = plsc.get_sparse_core_info()
    NW = info.num_cores * info.num_subcores
    nbg = N // _L
    n_groups = B * nbg
    base_loc = n_groups // NW
    n_rem = n_groups - base_loc * NW
    assert base_loc % 2 == 0, "pair-wise double buffering expects even count"

    S = _STAGES
    SLAB = S * _L * G
    RSLAB = S * _L * 4

    mesh = plsc.VectorSubcoreMesh(core_axis_name="c", subcore_axis_name="s")

    def body(boxes_in, gt_in, res_out, sims_out,
             gt_v, area_v, inbox, slab0, slab1, rslab0, rslab1, sem0, sem1):
        wid = lax.axis_index("s") * info.num_cores + lax.axis_index("c")
        pltpu.sync_copy(gt_in, gt_v)

        iota = lax.iota(jnp.int32, _L)
        col4 = iota * 4
        colG = iota * G

        for i in range(B * G // _L):
            ibase = col4 + (i * _L * 4)
            x1 = plsc.load_gather(gt_v, [ibase])
            y1 = plsc.load_gather(gt_v, [ibase + 1])
            x2 = plsc.load_gather(gt_v, [ibase + 2])
            y2 = plsc.load_gather(gt_v, [ibase + 3])
            area_v[pl.ds(i * _L, _L)] = (x2 - x1) * (y2 - y1)

        def drain(slab, rslab, sem):
            pltpu.make_async_copy(sims_out.at[pl.ds(0, SLAB)], slab, sem).wait()
            pltpu.make_async_copy(res_out.at[pl.ds(0, RSLAB)], rslab, sem).wait()

        def process_group(gid, slab, rslab, sem):
            b = (gid >= nbg).astype(jnp.int32)
            n0 = (gid - b * nbg) * _L
            off_in = (b * N + n0) * 4
            pltpu.sync_copy(boxes_in.at[pl.ds(off_in, _L * 4)], inbox)
            px1 = plsc.load_gather(inbox, [col4])
            py1 = plsc.load_gather(inbox, [col4 + 1])
            px2 = plsc.load_gather(inbox, [col4 + 2])
            py2 = plsc.load_gather(inbox, [col4 + 3])
            gofs = b * (G * 4)
            aofs = b * G
            cur = (px1, py1, px2, py2)
            for s in range(S):
                px1, py1, px2, py2 = cur
                plsc.store_scatter(rslab, [col4 + s * (_L * 4)], px1)
                plsc.store_scatter(rslab, [col4 + (s * (_L * 4) + 1)], py1)
                plsc.store_scatter(rslab, [col4 + (s * (_L * 4) + 2)], px2)
                plsc.store_scatter(rslab, [col4 + (s * (_L * 4) + 3)], py2)
                area_p = (px2 - px1) * (py2 - py1)
                sbase = colG + s * (_L * G)

                def gt_body(g, carry, _ap=area_p, _c=cur, _sb=sbase,
                            _gofs=gofs, _aofs=aofs, _slab=slab):
                    bv, bi = carry
                    _px1, _py1, _px2, _py2 = _c
                    gb = jnp.broadcast_to(_gofs + g * 4, (_L,))
                    gx1 = plsc.load_gather(gt_v, [gb])
                    gy1 = plsc.load_gather(gt_v, [gb + 1])
                    gx2 = plsc.load_gather(gt_v, [gb + 2])
                    gy2 = plsc.load_gather(gt_v, [gb + 3])
                    ag = plsc.load_gather(
                        area_v, [jnp.broadcast_to(_aofs + g, (_L,))])
                    w = jnp.maximum(
                        jnp.minimum(_px2, gx2) - jnp.maximum(_px1, gx1), 0.0)
                    h = jnp.maximum(
                        jnp.minimum(_py2, gy2) - jnp.maximum(_py1, gy1), 0.0)
                    inter = w * h
                    union = (_ap + ag) - inter
                    iou = inter / union
                    plsc.store_scatter(_slab, [_sb + g], iou)
                    m = iou > bv
                    bv = jnp.where(m, iou, bv)
                    bi = jnp.where(m, g, bi)
                    return bv, bi

                bv0 = jnp.full((_L,), -1.0, dtype=jnp.float32)
                bi0 = jnp.zeros((_L,), dtype=jnp.int32)
                _, bi = lax.fori_loop(0, G, gt_body, (bv0, bi0))

                if s < S - 1:
                    gidx = gofs + bi * 4
                    gx1 = plsc.load_gather(gt_v, [gidx])
                    gy1 = plsc.load_gather(gt_v, [gidx + 1])
                    gx2 = plsc.load_gather(gt_v, [gidx + 2])
                    gy2 = plsc.load_gather(gt_v, [gidx + 3])
                    dcx = (gx1 + gx2) / 2.0 - (px1 + px2) / 2.0
                    dcy = (gy1 + gy2) / 2.0 - (py1 + py2) / 2.0
                    dw = (gx2 - gx1) - (px2 - px1)
                    dh = (gy2 - gy1) - (py2 - py1)
                    npx1 = px1 + _LR_POS * dcx
                    npy1 = py1 + _LR_POS * dcy
                    npx2 = ((px2 + _LR_POS * dcx) + _LR_SIZE * dw) - _LR_SIZE * dcx
                    npy2 = ((py2 + _LR_POS * dcy) + _LR_SIZE * dh) - _LR_SIZE * dcy
                    cur = (npx1, npy1, npx2, npy2)

            for s in range(S):
                soff = (b * S + s) * (N * G) + n0 * G
                pltpu.async_copy(slab.at[pl.ds(s * _L * G, _L * G)],
                                 sims_out.at[pl.ds(soff, _L * G)], sem)
                roff = (b * S + s) * (N * 4) + n0 * 4
                pltpu.async_copy(rslab.at[pl.ds(s * _L * 4, _L * 4)],
                                 res_out.at[pl.ds(roff, _L * 4)], sem)

        def pair_body(j2, carry):
            gid_a = wid + NW * (2 * j2)

            @pl.when(j2 > 0)
            def _():
                drain(slab0, rslab0, sem0)

            process_group(gid_a, slab0, rslab0, sem0)

            @pl.when(j2 > 0)
            def _():
                drain(slab1, rslab1, sem1)

            process_group(gid_a + NW, slab1, rslab1, sem1)
            return carry

        lax.fori_loop(0, base_loc // 2, pair_body, 0)

        @pl.when(wid < n_rem)
        def _():
            drain(slab0, rslab0, sem0)
            process_group(wid + NW * base_loc, slab0, rslab0, sem0)

        drain(slab0, rslab0, sem0)
        drain(slab1, rslab1, sem1)

    return pl.kernel(
        body,
        out_type=(
            jax.ShapeDtypeStruct((B * S * N * 4,), jnp.float32),
            jax.ShapeDtypeStruct((B * S * N * G,), jnp.float32),
        ),
        mesh=mesh,
        compiler_params=pltpu.CompilerParams(needs_layout_passes=False),
        scratch_types=[
            pltpu.VMEM((B * G * 4,), jnp.float32),
            pltpu.VMEM((B * G,), jnp.float32),
            pltpu.VMEM((_L * 4,), jnp.float32),
            pltpu.VMEM((SLAB,), jnp.float32),
            pltpu.VMEM((SLAB,), jnp.float32),
            pltpu.VMEM((RSLAB,), jnp.float32),
            pltpu.VMEM((RSLAB,), jnp.float32),
            pltpu.SemaphoreType.DMA,
            pltpu.SemaphoreType.DMA,
        ],
    )


@jax.jit
def kernel(boxes, gt_boxes):
    B, N, _ = boxes.shape
    G = gt_boxes.shape[1]
    call = _make_sc_call(B, N, G)
    res, sims = call(boxes.reshape(-1), gt_boxes.reshape(-1))
    return (res.reshape(B, _STAGES, N, 4), sims.reshape(B, _STAGES, N, G))
